# Initial kernel scaffold; baseline (speedup 1.0000x reference)
import jax
import jax.numpy as jnp
from jax.experimental import pallas as pl


def kernel(geom_xyz, depth_features, context_features, batch_size, num_cams, num_depth, num_height, num_width):
    raise NotImplementedError("write your pallas kernel here")


# speedup vs baseline: 28.7369x; 28.7369x over previous
import functools

import jax
import jax.numpy as jnp
from jax import lax
from jax.experimental import pallas as pl
from jax.experimental.pallas import tpu as pltpu
from jax.experimental.pallas import tpu_sc as plsc

NUM_VOXEL_X = 128
NUM_VOXEL_Y = 128
NUM_CHANNELS = 80

NC = 2
NS = 16
L = 16


def _sc_pool(table, depth, geom_x, geom_y, *, num_cams, num_depth, hw):
    n_slices = num_cams * num_depth
    n_vox = NUM_VOXEL_Y * NUM_VOXEL_X
    CH = NUM_CHANNELS // NC
    slices_per_tile = n_slices // NS
    rows_per_tile = n_vox // NS
    n_chunks = hw // 64
    ch_offs = [cc * L for cc in range(CH // L)]
    if CH % L:
        ch_offs.append(CH - L)

    mesh = plsc.VectorSubcoreMesh(
        core_axis_name="c", subcore_axis_name="s", num_cores=NC, num_subcores=NS
    )

    @functools.partial(
        pl.kernel,
        out_type=jax.ShapeDtypeStruct((NC, n_vox, CH), jnp.float32),
        mesh=mesh,
        compiler_params=pltpu.CompilerParams(use_tc_tiling_on_sc=False),
        scratch_types=[
            pltpu.VMEM((hw, CH), jnp.float32),
            pltpu.VMEM((hw, CH), jnp.float32),
            pltpu.VMEM((hw,), jnp.float32),
            pltpu.VMEM((hw,), jnp.int32),
            pltpu.VMEM((hw,), jnp.int32),
            pltpu.VMEM((n_chunks, 64), jnp.int32),
            pltpu.VMEM_SHARED((n_vox, CH), jnp.float32),
        ],
    )
    def pool(table_hbm, depth_hbm, x_hbm, y_hbm, out_hbm,
             table_buf, res_buf, depth_buf, x_buf, y_buf, idx_buf, acc):
        cid = lax.axis_index("c")
        sid = lax.axis_index("s")

        def zero_row(r, _):
            z = jnp.zeros((L,), jnp.float32)
            for co in ch_offs:
                res_buf[r, pl.ds(co, L)] = z
            return 0

        lax.fori_loop(0, hw, zero_row, 0)
        base_row = sid * rows_per_tile
        pltpu.sync_copy(res_buf, acc.at[pl.ds(base_row, hw), :])
        pltpu.sync_copy(res_buf.at[pl.ds(0, rows_per_tile - hw), :],
                        acc.at[pl.ds(base_row + hw, rows_per_tile - hw), :])
        plsc.subcore_barrier()

        lo = sid * slices_per_tile

        def do_slice(i, prev_cam):
            s = lo + i
            cam = s // num_depth

            @pl.when(cam != prev_cam)
            def _():
                pltpu.sync_copy(
                    table_hbm.at[cid, pl.ds(cam * hw, hw), :], table_buf
                )

            pltpu.sync_copy(depth_hbm.at[pl.ds(s * hw, hw)], depth_buf)
            pltpu.sync_copy(x_hbm.at[pl.ds(s * hw, hw)], x_buf)
            pltpu.sync_copy(y_hbm.at[pl.ds(s * hw, hw)], y_buf)

            def mk_idx(j, _):
                for gg in range(4):
                    p0 = j * 64 + gg * L
                    x = x_buf[pl.ds(p0, L)]
                    y = y_buf[pl.ds(p0, L)]
                    x = jnp.minimum(jnp.maximum(x, 0), NUM_VOXEL_X - 1)
                    y = jnp.minimum(jnp.maximum(y, 0), NUM_VOXEL_Y - 1)
                    idx_buf[j, pl.ds(gg * L, L)] = y * NUM_VOXEL_X + x
                return 0

            lax.fori_loop(0, n_chunks, mk_idx, 0)

            def scale_rows(g, _):
                r0 = g * L
                dvec = depth_buf[pl.ds(r0, L)]
                for k in range(L):
                    dv = jnp.full((L,), dvec[k], jnp.float32)
                    for co in ch_offs:
                        res_buf[r0 + k, pl.ds(co, L)] = (
                            table_buf[r0 + k, pl.ds(co, L)] * dv
                        )
                return 0

            lax.fori_loop(0, hw // L, scale_rows, 0)

            def scat(j, _):
                pltpu.sync_copy(res_buf.at[pl.ds(j * 64, 64), :],
                                acc.at[idx_buf.at[j]], add=True)
                return 0

            lax.fori_loop(0, n_chunks, scat, 0)
            return cam

        lax.fori_loop(0, slices_per_tile, do_slice, jnp.int32(-1))

        plsc.subcore_barrier()
        pltpu.sync_copy(
            acc.at[pl.ds(base_row, rows_per_tile), :],
            out_hbm.at[cid, pl.ds(base_row, rows_per_tile), :],
        )

    return pool(table, depth, geom_x, geom_y)


def kernel(geom_xyz, depth_features, context_features,
           batch_size, num_cams, num_depth, num_height, num_width):
    _, cams, _, nh, nw = context_features.shape
    hw = nh * nw
    nd = geom_xyz.shape[0] // (cams * hw)
    table = jnp.transpose(context_features[0], (0, 2, 3, 1)).reshape(
        cams * hw, NUM_CHANNELS
    ).astype(jnp.float32)
    ch = NUM_CHANNELS // NC
    table2 = jnp.stack([table[:, c * ch:(c + 1) * ch] for c in range(NC)])
    depth = depth_features.astype(jnp.float32)
    geom_i = geom_xyz.astype(jnp.int32)
    geom_x = geom_i[:, 0]
    geom_y = geom_i[:, 1]

    parts = _sc_pool(table2, depth, geom_x, geom_y,
                     num_cams=cams, num_depth=nd, hw=hw)
    out = jnp.concatenate([parts[c] for c in range(NC)], axis=1)
    return out.reshape(context_features.shape[0], NUM_VOXEL_Y, NUM_VOXEL_X,
                       NUM_CHANNELS)
